# baseline (speedup 1.0000x reference)
import functools
import jax
import jax.numpy as jnp
from jax import lax
from jax.experimental import pallas as pl
from jax.experimental.pallas import tpu as pltpu

HIDDEN = 1024
D_FF = 4096
NUM_EXPERTS = 8
T = 2048
RBT = 256
FBT = 256
NBLK_EXP = T * 2 // FBT + NUM_EXPERTS
NBLK_SH = T // FBT
NBLK = NBLK_SH + NBLK_EXP
NSLOT = NBLK * FBT
SHBASE = NBLK_SH * FBT


def _route_body(x_ref, gw_ref, gb_ref,
                sA_ref, sB_ref, posA_ref, posB_ref, binfo_ref, nact_ref,
                carry, idxA_s, idxB_s, rankA_s, rankB_s):
    b = pl.program_id(0)
    nb_ = pl.num_programs(0)

    @pl.when(b == 0)
    def _():
        carry[...] = jnp.zeros_like(carry)

    logits = jnp.dot(x_ref[...], gw_ref[...], preferred_element_type=jnp.float32)
    scores = jax.nn.sigmoid(logits + gb_ref[...])
    iota = lax.broadcasted_iota(jnp.int32, (RBT, 16), 1)
    m1 = jnp.max(scores, axis=1, keepdims=True)
    idxA = jnp.min(jnp.where(scores == m1, iota, 99), axis=1, keepdims=True)
    scores2 = jnp.where(iota == idxA, -1.0, scores)
    m2 = jnp.max(scores2, axis=1, keepdims=True)
    idxB = jnp.min(jnp.where(scores2 == m2, iota, 99), axis=1, keepdims=True)
    denom = m1 + m2 + 1e-6

    ohA = (iota == idxA).astype(jnp.float32)
    ohB = (iota == idxB).astype(jnp.float32)
    ltm = (lax.broadcasted_iota(jnp.int32, (RBT, RBT), 0)
           > lax.broadcasted_iota(jnp.int32, (RBT, RBT), 1)).astype(jnp.float32)
    prefA = jnp.dot(ltm, ohA, preferred_element_type=jnp.float32) + carry[...]
    sumA = jnp.sum(ohA, axis=0, keepdims=True)
    prefB = jnp.dot(ltm, ohB, preferred_element_type=jnp.float32) + sumA + carry[...]
    rankA = jnp.sum(prefA * ohA, axis=1, keepdims=True)
    rankB = jnp.sum(prefB * ohB, axis=1, keepdims=True)
    carry[...] = carry[...] + sumA + jnp.sum(ohB, axis=0, keepdims=True)

    sl = pl.ds(b * RBT, RBT)
    idxA_s[sl, :] = idxA
    idxB_s[sl, :] = idxB
    rankA_s[sl, :] = rankA
    rankB_s[sl, :] = rankB
    sA_ref[...] = m1 / denom
    sB_ref[...] = m2 / denom

    @pl.when(b == nb_ - 1)
    def _():
        counts = carry[...]
        nblocks = jnp.floor((counts + (FBT - 1)) / FBT)
        cp = nblocks * FBT
        iota_r = lax.broadcasted_iota(jnp.int32, (16, 16), 0)
        iota_c = lax.broadcasted_iota(jnp.int32, (16, 16), 1)
        ut = (iota_r < iota_c).astype(jnp.float32)
        poff = jnp.dot(cp, ut, preferred_element_type=jnp.float32)

        iota2 = lax.broadcasted_iota(jnp.int32, (T, 16), 1)
        ohAs = (iota2 == idxA_s[...]).astype(jnp.float32)
        ohBs = (iota2 == idxB_s[...]).astype(jnp.float32)
        poA = jnp.sum(ohAs * poff, axis=1, keepdims=True)
        poB = jnp.sum(ohBs * poff, axis=1, keepdims=True)
        posA_ref[...] = (poA + rankA_s[...] + float(SHBASE)).astype(jnp.int32)
        posB_ref[...] = (poB + rankB_s[...] + float(SHBASE)).astype(jnp.int32)

        eye = (iota_r == iota_c).astype(jnp.float32)
        poff_col = lax.dot_general(eye, poff, (((1,), (1,)), ((), ())),
                                   preferred_element_type=jnp.float32)
        blk = lax.broadcasted_iota(jnp.int32, (1, 128), 1)
        r = ((blk - NBLK_SH) * FBT).astype(jnp.float32)
        a = (poff_col <= r).astype(jnp.float32)
        ebx = jnp.clip(jnp.sum(a, axis=0, keepdims=True) - 1.0,
                       0.0, float(NUM_EXPERTS - 1))
        bexp = jnp.where(blk < NBLK_SH, float(NUM_EXPERTS), ebx)
        binfo_ref[...] = bexp.astype(jnp.int32)
        total_blk = jnp.sum(nblocks, axis=1, keepdims=True) + float(NBLK_SH)
        nact_ref[...] = jnp.broadcast_to(total_blk, (1, 128)).astype(jnp.int32)


def _ffn_body(binfo_ref, nact_ref, posA_ref, posB_ref, sA_ref, sB_ref, xbf_ref,
              w1_ref, b1_ref, w2_ref, b2_ref, y_ref):
    b = pl.program_id(0)

    @pl.when(b < nact_ref[0])
    def _():
        slot = b * FBT + lax.broadcasted_iota(jnp.int32, (1, FBT), 1)
        tok_col = lax.broadcasted_iota(jnp.int32, (T, 1), 0)
        eq_sh = (tok_col == slot)
        eqA = posA_ref[...] == slot
        eqB = posB_ref[...] == slot
        m = jnp.where(b < NBLK_SH, eq_sh.astype(jnp.bfloat16),
                      (eqA | eqB).astype(jnp.bfloat16))
        x = lax.dot_general(m, xbf_ref[...], (((0,), (0,)), ((), ())),
                            preferred_element_type=jnp.float32)
        x = x.astype(jnp.bfloat16)
        wm = (eqA.astype(jnp.float32) * sA_ref[...]
              + eqB.astype(jnp.float32) * sB_ref[...])
        wcol = lax.dot_general(wm, jnp.ones((T, 1), jnp.float32),
                               (((0,), (0,)), ((), ())),
                               precision=lax.Precision.HIGHEST,
                               preferred_element_type=jnp.float32)
        wcol = jnp.where(b < NBLK_SH, 0.1, wcol)
        h = jnp.dot(x, w1_ref[...], preferred_element_type=jnp.float32)
        h = jax.nn.gelu(h + b1_ref[0])
        y = jnp.dot(h.astype(jnp.bfloat16), w2_ref[...],
                    preferred_element_type=jnp.float32) + b2_ref[0]
        y_ref[...] = y * wcol


def _make_sc_combine():
    from jax.experimental.pallas import tpu_sc as plsc
    info = plsc.get_sparse_core_info()
    NC, NS, L = info.num_cores, info.num_subcores, info.num_lanes
    NW = NC * NS
    TPS = T // NW
    CH = L
    mesh = plsc.VectorSubcoreMesh(core_axis_name="c", subcore_axis_name="s")

    @functools.partial(
        pl.kernel, mesh=mesh,
        out_type=jax.ShapeDtypeStruct((T, HIDDEN), jnp.float32),
        scratch_types=[
            pltpu.VMEM((TPS, HIDDEN), jnp.float32),
            pltpu.VMEM((CH, HIDDEN), jnp.float32),
            pltpu.VMEM((CH,), jnp.int32),
            pltpu.SemaphoreType.DMA,
        ],
    )
    def k56(y_hbm, posA_hbm, posB_hbm, out_hbm, acc, v, iv, sem):
        wid = lax.axis_index("s") * NC + lax.axis_index("c")
        tok0 = wid * TPS
        pltpu.sync_copy(y_hbm.at[pl.ds(tok0, TPS)], acc)
        for c in range(TPS // CH):
            for pos_hbm in (posA_hbm, posB_hbm):
                pltpu.sync_copy(pos_hbm.at[pl.ds(tok0 + c * CH, CH)], iv)
                pltpu.async_copy(y_hbm.at[iv], v, sem).wait()

                def tok_body(i, _, _c=c):
                    row = _c * CH + i
                    for j in range(HIDDEN // L):
                        sl = pl.ds(j * L, L)
                        acc[row, sl] = acc[row, sl] + v[i, sl]
                    return 0

                lax.fori_loop(0, CH, tok_body, 0)
        pltpu.sync_copy(acc, out_hbm.at[pl.ds(tok0, TPS)])

    return k56


def kernel(x, gate_w, gate_bias, W1, b1, W2, b2, Ws1, bs1, Ws2, bs2):
    B, S, H = x.shape
    xf = x.reshape(-1, H)

    gw16 = jnp.pad(gate_w, ((0, 0), (0, 16 - NUM_EXPERTS)))
    gb16 = jnp.pad(gate_bias, (0, 16 - NUM_EXPERTS),
                   constant_values=-1e9).reshape(1, 16)

    nblk_r = T // RBT
    sA, sB, posA, posB, binfo, nact = pl.pallas_call(
        _route_body,
        grid=(nblk_r,),
        in_specs=[
            pl.BlockSpec((RBT, HIDDEN), lambda b: (b, 0)),
            pl.BlockSpec((HIDDEN, 16), lambda b: (0, 0)),
            pl.BlockSpec((1, 16), lambda b: (0, 0)),
        ],
        out_specs=[
            pl.BlockSpec((RBT, 1), lambda b: (b, 0)),
            pl.BlockSpec((RBT, 1), lambda b: (b, 0)),
            pl.BlockSpec((T, 1), lambda b: (0, 0)),
            pl.BlockSpec((T, 1), lambda b: (0, 0)),
            pl.BlockSpec((1, 128), lambda b: (0, 0)),
            pl.BlockSpec((1, 128), lambda b: (0, 0)),
        ],
        out_shape=[
            jax.ShapeDtypeStruct((T, 1), jnp.float32),
            jax.ShapeDtypeStruct((T, 1), jnp.float32),
            jax.ShapeDtypeStruct((T, 1), jnp.int32),
            jax.ShapeDtypeStruct((T, 1), jnp.int32),
            jax.ShapeDtypeStruct((1, 128), jnp.int32),
            jax.ShapeDtypeStruct((1, 128), jnp.int32),
        ],
        scratch_shapes=[pltpu.VMEM((1, 16), jnp.float32),
                        pltpu.VMEM((T, 1), jnp.int32),
                        pltpu.VMEM((T, 1), jnp.int32),
                        pltpu.VMEM((T, 1), jnp.float32),
                        pltpu.VMEM((T, 1), jnp.float32)],
    )(xf, gw16, gb16)

    W1s = jnp.concatenate([W1, Ws1[None]], axis=0).astype(jnp.bfloat16)
    W1s = W1s.reshape((NUM_EXPERTS + 1) * HIDDEN, D_FF)
    W2s = jnp.concatenate([W2, Ws2[None]], axis=0).astype(jnp.bfloat16)
    W2s = W2s.reshape((NUM_EXPERTS + 1) * D_FF, HIDDEN)
    b1s = jnp.concatenate([b1, bs1[None]], axis=0).reshape(NUM_EXPERTS + 1, 1, D_FF)
    b2s = jnp.concatenate([b2, bs2[None]], axis=0).reshape(NUM_EXPERTS + 1, 1, HIDDEN)
    xbf = xf.astype(jnp.bfloat16)

    grid_spec = pltpu.PrefetchScalarGridSpec(
        num_scalar_prefetch=2,
        grid=(NBLK,),
        in_specs=[
            pl.BlockSpec((T, 1), lambda b, binfo, nact: (0, 0)),
            pl.BlockSpec((T, 1), lambda b, binfo, nact: (0, 0)),
            pl.BlockSpec((T, 1), lambda b, binfo, nact: (0, 0)),
            pl.BlockSpec((T, 1), lambda b, binfo, nact: (0, 0)),
            pl.BlockSpec((T, HIDDEN), lambda b, binfo, nact: (0, 0)),
            pl.BlockSpec((HIDDEN, D_FF), lambda b, binfo, nact: (binfo[b], 0)),
            pl.BlockSpec((1, 1, D_FF), lambda b, binfo, nact: (binfo[b], 0, 0)),
            pl.BlockSpec((D_FF, HIDDEN), lambda b, binfo, nact: (binfo[b], 0)),
            pl.BlockSpec((1, 1, HIDDEN), lambda b, binfo, nact: (binfo[b], 0, 0)),
        ],
        out_specs=pl.BlockSpec((FBT, HIDDEN), lambda b, binfo, nact: (b, 0)),
    )
    y = pl.pallas_call(
        _ffn_body,
        grid_spec=grid_spec,
        out_shape=jax.ShapeDtypeStruct((NSLOT, HIDDEN), jnp.float32),
    )(binfo.reshape(128), nact.reshape(128)[:1], posA, posB, sA, sB, xbf,
      W1s, b1s, W2s, b2s)

    pos0 = posA.reshape(T)
    pos1 = posB.reshape(T)
    out = _make_sc_combine()(y, pos0, pos1)

    return out.reshape(B, S, H)


# speedup vs baseline: 1.1393x; 1.1393x over previous
import functools
import jax
import jax.numpy as jnp
from jax import lax
from jax.experimental import pallas as pl
from jax.experimental.pallas import tpu as pltpu

HIDDEN = 1024
D_FF = 4096
NUM_EXPERTS = 8
T = 2048
RBT = 256
FBT = 256
NBLK_EXP = T * 2 // FBT + NUM_EXPERTS
NBLK_SH = T // FBT
NBLK = NBLK_SH + NBLK_EXP
NSLOT = NBLK * FBT
SHBASE = NBLK_SH * FBT


def _route_body(x_ref, gw_ref, gb_ref,
                sA_ref, sB_ref, posA_ref, posB_ref, binfo_ref, nact_ref,
                carry, idxA_s, idxB_s, rankA_s, rankB_s):
    b = pl.program_id(0)
    nb_ = pl.num_programs(0)

    @pl.when(b == 0)
    def _():
        carry[...] = jnp.zeros_like(carry)

    logits = jnp.dot(x_ref[...], gw_ref[...], preferred_element_type=jnp.float32)
    scores = jax.nn.sigmoid(logits + gb_ref[...])
    iota = lax.broadcasted_iota(jnp.int32, (RBT, 16), 1)
    m1 = jnp.max(scores, axis=1, keepdims=True)
    idxA = jnp.min(jnp.where(scores == m1, iota, 99), axis=1, keepdims=True)
    scores2 = jnp.where(iota == idxA, -1.0, scores)
    m2 = jnp.max(scores2, axis=1, keepdims=True)
    idxB = jnp.min(jnp.where(scores2 == m2, iota, 99), axis=1, keepdims=True)
    denom = m1 + m2 + 1e-6

    ohA = (iota == idxA).astype(jnp.float32)
    ohB = (iota == idxB).astype(jnp.float32)
    ltm = (lax.broadcasted_iota(jnp.int32, (RBT, RBT), 0)
           > lax.broadcasted_iota(jnp.int32, (RBT, RBT), 1)).astype(jnp.float32)
    prefA = jnp.dot(ltm, ohA, preferred_element_type=jnp.float32) + carry[...]
    sumA = jnp.sum(ohA, axis=0, keepdims=True)
    prefB = jnp.dot(ltm, ohB, preferred_element_type=jnp.float32) + sumA + carry[...]
    rankA = jnp.sum(prefA * ohA, axis=1, keepdims=True)
    rankB = jnp.sum(prefB * ohB, axis=1, keepdims=True)
    carry[...] = carry[...] + sumA + jnp.sum(ohB, axis=0, keepdims=True)

    sl = pl.ds(b * RBT, RBT)
    idxA_s[sl, :] = idxA
    idxB_s[sl, :] = idxB
    rankA_s[sl, :] = rankA
    rankB_s[sl, :] = rankB
    sA_ref[...] = m1 / denom
    sB_ref[...] = m2 / denom

    @pl.when(b == nb_ - 1)
    def _():
        counts = carry[...]
        nblocks = jnp.floor((counts + (FBT - 1)) / FBT)
        cp = nblocks * FBT
        iota_r = lax.broadcasted_iota(jnp.int32, (16, 16), 0)
        iota_c = lax.broadcasted_iota(jnp.int32, (16, 16), 1)
        ut = (iota_r < iota_c).astype(jnp.float32)
        poff = jnp.dot(cp, ut, preferred_element_type=jnp.float32)

        iota2 = lax.broadcasted_iota(jnp.int32, (T, 16), 1)
        ohAs = (iota2 == idxA_s[...]).astype(jnp.float32)
        ohBs = (iota2 == idxB_s[...]).astype(jnp.float32)
        poA = jnp.sum(ohAs * poff, axis=1, keepdims=True)
        poB = jnp.sum(ohBs * poff, axis=1, keepdims=True)
        posA_ref[...] = (poA + rankA_s[...] + float(SHBASE)).astype(jnp.int32)
        posB_ref[...] = (poB + rankB_s[...] + float(SHBASE)).astype(jnp.int32)

        eye = (iota_r == iota_c).astype(jnp.float32)
        poff_col = lax.dot_general(eye, poff, (((1,), (1,)), ((), ())),
                                   preferred_element_type=jnp.float32)
        blk = lax.broadcasted_iota(jnp.int32, (1, 128), 1)
        r = ((blk - NBLK_SH) * FBT).astype(jnp.float32)
        a = (poff_col <= r).astype(jnp.float32)
        ebx = jnp.clip(jnp.sum(a, axis=0, keepdims=True) - 1.0,
                       0.0, float(NUM_EXPERTS - 1))
        bexp = jnp.where(blk < NBLK_SH, float(NUM_EXPERTS), ebx)
        binfo_ref[...] = bexp.astype(jnp.int32)
        total_blk = jnp.sum(nblocks, axis=1, keepdims=True) + float(NBLK_SH)
        nact_ref[...] = jnp.broadcast_to(total_blk, (1, 128)).astype(jnp.int32)


def _ffn_body(binfo_ref, nact_ref, xs_ref, ws_ref, w1_ref, b1_ref, w2_ref,
              b2_ref, y_ref):
    b = pl.program_id(0)

    @pl.when(b < nact_ref[0])
    def _():
        x = xs_ref[...].astype(jnp.bfloat16)
        h = jnp.dot(x, w1_ref[...], preferred_element_type=jnp.float32)
        h = jax.nn.gelu(h + b1_ref[0])
        y = jnp.dot(h.astype(jnp.bfloat16), w2_ref[...],
                    preferred_element_type=jnp.float32) + b2_ref[0]
        wcol = jnp.where(b < NBLK_SH, 0.1, ws_ref[:, :1])
        y_ref[...] = y * wcol


def _make_sc_scatter():
    from jax.experimental.pallas import tpu_sc as plsc
    info = plsc.get_sparse_core_info()
    NC, NS = info.num_cores, info.num_subcores
    NW = NC * NS
    rows_per_w = T // NW
    mesh = plsc.VectorSubcoreMesh(core_axis_name="c", subcore_axis_name="s")

    @functools.partial(
        pl.kernel, mesh=mesh,
        out_type=[jax.ShapeDtypeStruct((NSLOT, HIDDEN), jnp.float32),
                  jax.ShapeDtypeStruct((NSLOT, 128), jnp.float32)],
        scratch_types=[
            pltpu.VMEM((rows_per_w, HIDDEN), jnp.float32),
            pltpu.VMEM((rows_per_w, 128), jnp.float32),
            pltpu.VMEM((rows_per_w, 128), jnp.float32),
            pltpu.VMEM((rows_per_w,), jnp.int32),
            pltpu.VMEM((rows_per_w,), jnp.int32),
            pltpu.SemaphoreType.DMA,
        ],
    )
    def k3(x_hbm, posA_hbm, posB_hbm, sA_hbm, sB_hbm, xs_hbm, ws_hbm,
           xv, w0v, w1v, i0v, i1v, sem):
        wid = lax.axis_index("s") * NC + lax.axis_index("c")
        base = wid * rows_per_w
        pltpu.sync_copy(posA_hbm.at[pl.ds(base, rows_per_w)], i0v)
        pltpu.sync_copy(posB_hbm.at[pl.ds(base, rows_per_w)], i1v)
        pltpu.sync_copy(x_hbm.at[pl.ds(base, rows_per_w)], xv)
        pltpu.sync_copy(sA_hbm.at[pl.ds(base, rows_per_w)], w0v)
        pltpu.sync_copy(sB_hbm.at[pl.ds(base, rows_per_w)], w1v)
        c1 = pltpu.async_copy(xv, xs_hbm.at[pl.ds(base, rows_per_w)], sem)
        c2 = pltpu.async_copy(xv, xs_hbm.at[i0v], sem)
        c3 = pltpu.async_copy(xv, xs_hbm.at[i1v], sem)
        c4 = pltpu.async_copy(w0v, ws_hbm.at[i0v], sem)
        c5 = pltpu.async_copy(w1v, ws_hbm.at[i1v], sem)
        c1.wait()
        c2.wait()
        c3.wait()
        c4.wait()
        c5.wait()

    return k3


def _make_sc_combine():
    from jax.experimental.pallas import tpu_sc as plsc
    info = plsc.get_sparse_core_info()
    NC, NS, L = info.num_cores, info.num_subcores, info.num_lanes
    NW = NC * NS
    TPS = T // NW
    CH = L
    mesh = plsc.VectorSubcoreMesh(core_axis_name="c", subcore_axis_name="s")

    @functools.partial(
        pl.kernel, mesh=mesh,
        out_type=jax.ShapeDtypeStruct((T, HIDDEN), jnp.float32),
        scratch_types=[
            pltpu.VMEM((TPS, HIDDEN), jnp.float32),
            pltpu.VMEM((CH, HIDDEN), jnp.float32),
            pltpu.VMEM((CH,), jnp.int32),
            pltpu.SemaphoreType.DMA,
        ],
    )
    def k56(y_hbm, posA_hbm, posB_hbm, out_hbm, acc, v, iv, sem):
        wid = lax.axis_index("s") * NC + lax.axis_index("c")
        tok0 = wid * TPS
        pltpu.sync_copy(y_hbm.at[pl.ds(tok0, TPS)], acc)
        for c in range(TPS // CH):
            for pos_hbm in (posA_hbm, posB_hbm):
                pltpu.sync_copy(pos_hbm.at[pl.ds(tok0 + c * CH, CH)], iv)
                pltpu.async_copy(y_hbm.at[iv], v, sem).wait()

                def tok_body(i, _, _c=c):
                    row = _c * CH + i
                    for j in range(HIDDEN // L):
                        sl = pl.ds(j * L, L)
                        acc[row, sl] = acc[row, sl] + v[i, sl]
                    return 0

                lax.fori_loop(0, CH, tok_body, 0)
        pltpu.sync_copy(acc, out_hbm.at[pl.ds(tok0, TPS)])

    return k56


def kernel(x, gate_w, gate_bias, W1, b1, W2, b2, Ws1, bs1, Ws2, bs2):
    B, S, H = x.shape
    xf = x.reshape(-1, H)

    gw16 = jnp.pad(gate_w, ((0, 0), (0, 16 - NUM_EXPERTS)))
    gb16 = jnp.pad(gate_bias, (0, 16 - NUM_EXPERTS),
                   constant_values=-1e9).reshape(1, 16)

    nblk_r = T // RBT
    sA, sB, posA, posB, binfo, nact = pl.pallas_call(
        _route_body,
        grid=(nblk_r,),
        in_specs=[
            pl.BlockSpec((RBT, HIDDEN), lambda b: (b, 0)),
            pl.BlockSpec((HIDDEN, 16), lambda b: (0, 0)),
            pl.BlockSpec((1, 16), lambda b: (0, 0)),
        ],
        out_specs=[
            pl.BlockSpec((RBT, 1), lambda b: (b, 0)),
            pl.BlockSpec((RBT, 1), lambda b: (b, 0)),
            pl.BlockSpec((T, 1), lambda b: (0, 0)),
            pl.BlockSpec((T, 1), lambda b: (0, 0)),
            pl.BlockSpec((1, 128), lambda b: (0, 0)),
            pl.BlockSpec((1, 128), lambda b: (0, 0)),
        ],
        out_shape=[
            jax.ShapeDtypeStruct((T, 1), jnp.float32),
            jax.ShapeDtypeStruct((T, 1), jnp.float32),
            jax.ShapeDtypeStruct((T, 1), jnp.int32),
            jax.ShapeDtypeStruct((T, 1), jnp.int32),
            jax.ShapeDtypeStruct((1, 128), jnp.int32),
            jax.ShapeDtypeStruct((1, 128), jnp.int32),
        ],
        scratch_shapes=[pltpu.VMEM((1, 16), jnp.float32),
                        pltpu.VMEM((T, 1), jnp.int32),
                        pltpu.VMEM((T, 1), jnp.int32),
                        pltpu.VMEM((T, 1), jnp.float32),
                        pltpu.VMEM((T, 1), jnp.float32)],
    )(xf, gw16, gb16)

    pos0 = posA.reshape(T)
    pos1 = posB.reshape(T)

    s0b = jnp.broadcast_to(sA, (T, 128))
    s1b = jnp.broadcast_to(sB, (T, 128))
    xs, wsort = _make_sc_scatter()(xf, pos0, pos1, s0b, s1b)

    W1s = jnp.concatenate([W1, Ws1[None]], axis=0).astype(jnp.bfloat16)
    W1s = W1s.reshape((NUM_EXPERTS + 1) * HIDDEN, D_FF)
    W2s = jnp.concatenate([W2, Ws2[None]], axis=0).astype(jnp.bfloat16)
    W2s = W2s.reshape((NUM_EXPERTS + 1) * D_FF, HIDDEN)
    b1s = jnp.concatenate([b1, bs1[None]], axis=0).reshape(NUM_EXPERTS + 1, 1, D_FF)
    b2s = jnp.concatenate([b2, bs2[None]], axis=0).reshape(NUM_EXPERTS + 1, 1, HIDDEN)

    grid_spec = pltpu.PrefetchScalarGridSpec(
        num_scalar_prefetch=2,
        grid=(NBLK,),
        in_specs=[
            pl.BlockSpec((FBT, HIDDEN), lambda b, binfo, nact: (b, 0)),
            pl.BlockSpec((FBT, 128), lambda b, binfo, nact: (b, 0)),
            pl.BlockSpec((HIDDEN, D_FF), lambda b, binfo, nact: (binfo[b], 0)),
            pl.BlockSpec((1, 1, D_FF), lambda b, binfo, nact: (binfo[b], 0, 0)),
            pl.BlockSpec((D_FF, HIDDEN), lambda b, binfo, nact: (binfo[b], 0)),
            pl.BlockSpec((1, 1, HIDDEN), lambda b, binfo, nact: (binfo[b], 0, 0)),
        ],
        out_specs=pl.BlockSpec((FBT, HIDDEN), lambda b, binfo, nact: (b, 0)),
    )
    y = pl.pallas_call(
        _ffn_body,
        grid_spec=grid_spec,
        out_shape=jax.ShapeDtypeStruct((NSLOT, HIDDEN), jnp.float32),
    )(binfo.reshape(128), nact.reshape(128)[:1], xs, wsort, W1s, b1s, W2s, b2s)

    out = _make_sc_combine()(y, pos0, pos1)

    return out.reshape(B, S, H)


# speedup vs baseline: 1.1959x; 1.0497x over previous
import functools
import jax
import jax.numpy as jnp
from jax import lax
from jax.experimental import pallas as pl
from jax.experimental.pallas import tpu as pltpu

HIDDEN = 1024
D_FF = 4096
NUM_EXPERTS = 8
T = 2048
RBT = 256
FBT = 256
NBLK_EXP = T * 2 // FBT + NUM_EXPERTS
NBLK_SH = T // FBT
NBLK = NBLK_SH + NBLK_EXP
NSLOT = NBLK * FBT
SHBASE = NBLK_SH * FBT


def _route_body(x_ref, gw_ref, gb_ref,
                sA_ref, sB_ref, posA_ref, posB_ref, binfo_ref, nact_ref,
                carry, idxA_s, idxB_s, rankA_s, rankB_s):
    b = pl.program_id(0)
    nb_ = pl.num_programs(0)

    @pl.when(b == 0)
    def _():
        carry[...] = jnp.zeros_like(carry)

    logits = jnp.dot(x_ref[...], gw_ref[...], preferred_element_type=jnp.float32)
    scores = jax.nn.sigmoid(logits + gb_ref[...])
    iota = lax.broadcasted_iota(jnp.int32, (RBT, 16), 1)
    m1 = jnp.max(scores, axis=1, keepdims=True)
    idxA = jnp.min(jnp.where(scores == m1, iota, 99), axis=1, keepdims=True)
    scores2 = jnp.where(iota == idxA, -1.0, scores)
    m2 = jnp.max(scores2, axis=1, keepdims=True)
    idxB = jnp.min(jnp.where(scores2 == m2, iota, 99), axis=1, keepdims=True)
    denom = m1 + m2 + 1e-6

    ohA = (iota == idxA).astype(jnp.float32)
    ohB = (iota == idxB).astype(jnp.float32)
    ltm = (lax.broadcasted_iota(jnp.int32, (RBT, RBT), 0)
           > lax.broadcasted_iota(jnp.int32, (RBT, RBT), 1)).astype(jnp.float32)
    prefA = jnp.dot(ltm, ohA, preferred_element_type=jnp.float32) + carry[...]
    sumA = jnp.sum(ohA, axis=0, keepdims=True)
    prefB = jnp.dot(ltm, ohB, preferred_element_type=jnp.float32) + sumA + carry[...]
    rankA = jnp.sum(prefA * ohA, axis=1, keepdims=True)
    rankB = jnp.sum(prefB * ohB, axis=1, keepdims=True)
    carry[...] = carry[...] + sumA + jnp.sum(ohB, axis=0, keepdims=True)

    sl = pl.ds(b * RBT, RBT)
    idxA_s[sl, :] = idxA
    idxB_s[sl, :] = idxB
    rankA_s[sl, :] = rankA
    rankB_s[sl, :] = rankB
    sA_ref[...] = m1 / denom
    sB_ref[...] = m2 / denom

    @pl.when(b == nb_ - 1)
    def _():
        counts = carry[...]
        nblocks = jnp.floor((counts + (FBT - 1)) / FBT)
        cp = nblocks * FBT
        iota_r = lax.broadcasted_iota(jnp.int32, (16, 16), 0)
        iota_c = lax.broadcasted_iota(jnp.int32, (16, 16), 1)
        ut = (iota_r < iota_c).astype(jnp.float32)
        poff = jnp.dot(cp, ut, preferred_element_type=jnp.float32)

        iota2 = lax.broadcasted_iota(jnp.int32, (T, 16), 1)
        ohAs = (iota2 == idxA_s[...]).astype(jnp.float32)
        ohBs = (iota2 == idxB_s[...]).astype(jnp.float32)
        poA = jnp.sum(ohAs * poff, axis=1, keepdims=True)
        poB = jnp.sum(ohBs * poff, axis=1, keepdims=True)
        posA_ref[...] = (poA + rankA_s[...] + float(SHBASE)).astype(jnp.int32)
        posB_ref[...] = (poB + rankB_s[...] + float(SHBASE)).astype(jnp.int32)

        eye = (iota_r == iota_c).astype(jnp.float32)
        poff_col = lax.dot_general(eye, poff, (((1,), (1,)), ((), ())),
                                   preferred_element_type=jnp.float32)
        blk = lax.broadcasted_iota(jnp.int32, (1, 128), 1)
        r = ((blk - NBLK_SH) * FBT).astype(jnp.float32)
        a = (poff_col <= r).astype(jnp.float32)
        ebx = jnp.clip(jnp.sum(a, axis=0, keepdims=True) - 1.0,
                       0.0, float(NUM_EXPERTS - 1))
        bexp = jnp.where(blk < NBLK_SH, float(NUM_EXPERTS), ebx)
        binfo_ref[...] = bexp.astype(jnp.int32)
        total_blk = jnp.sum(nblocks, axis=1, keepdims=True) + float(NBLK_SH)
        nact_ref[...] = jnp.broadcast_to(total_blk, (1, 128)).astype(jnp.int32)


def _ffn_body(binfo_ref, nact_ref, xs_ref, w1_ref, b1_ref, w2_ref, b2_ref, y_ref):
    b = pl.program_id(0)

    @pl.when(b < nact_ref[0])
    def _():
        x = xs_ref[...].astype(jnp.bfloat16)
        h = jnp.dot(x, w1_ref[...], preferred_element_type=jnp.float32)
        h = jax.nn.gelu(h + b1_ref[0])
        y = jnp.dot(h.astype(jnp.bfloat16), w2_ref[...],
                    preferred_element_type=jnp.float32) + b2_ref[0]
        y_ref[...] = y


def _combine_body(y0_ref, y1_ref, ysh_ref, s0_ref, s1_ref, o_ref):
    o_ref[...] = (s0_ref[...] * y0_ref[...] + s1_ref[...] * y1_ref[...]
                  + 0.1 * ysh_ref[...])


def _make_sc_scatter():
    from jax.experimental.pallas import tpu_sc as plsc
    info = plsc.get_sparse_core_info()
    NC, NS = info.num_cores, info.num_subcores
    NW = NC * NS
    rows_per_w = T // NW
    mesh = plsc.VectorSubcoreMesh(core_axis_name="c", subcore_axis_name="s")

    @functools.partial(
        pl.kernel, mesh=mesh,
        out_type=jax.ShapeDtypeStruct((NSLOT, HIDDEN), jnp.float32),
        scratch_types=[
            pltpu.VMEM((rows_per_w, HIDDEN), jnp.float32),
            pltpu.VMEM((rows_per_w,), jnp.int32),
            pltpu.VMEM((rows_per_w,), jnp.int32),
            pltpu.SemaphoreType.DMA,
        ],
    )
    def k3(x_hbm, posA_hbm, posB_hbm, xs_hbm, xv, i0v, i1v, sem):
        wid = lax.axis_index("s") * NC + lax.axis_index("c")
        base = wid * rows_per_w
        pltpu.sync_copy(posA_hbm.at[pl.ds(base, rows_per_w)], i0v)
        pltpu.sync_copy(posB_hbm.at[pl.ds(base, rows_per_w)], i1v)
        pltpu.sync_copy(x_hbm.at[pl.ds(base, rows_per_w)], xv)
        c1 = pltpu.async_copy(xv, xs_hbm.at[pl.ds(base, rows_per_w)], sem)
        c2 = pltpu.async_copy(xv, xs_hbm.at[i0v], sem)
        c3 = pltpu.async_copy(xv, xs_hbm.at[i1v], sem)
        c1.wait()
        c2.wait()
        c3.wait()

    return k3


def _make_sc_gather():
    from jax.experimental.pallas import tpu_sc as plsc
    info = plsc.get_sparse_core_info()
    NC, NS = info.num_cores, info.num_subcores
    NW = NC * NS
    rows_per_w = T // NW
    mesh = plsc.VectorSubcoreMesh(core_axis_name="c", subcore_axis_name="s")

    @functools.partial(
        pl.kernel, mesh=mesh,
        out_type=[jax.ShapeDtypeStruct((T, HIDDEN), jnp.float32),
                  jax.ShapeDtypeStruct((T, HIDDEN), jnp.float32)],
        scratch_types=[
            pltpu.VMEM((rows_per_w // 2, HIDDEN), jnp.float32),
            pltpu.VMEM((rows_per_w // 2, HIDDEN), jnp.float32),
            pltpu.VMEM((rows_per_w // 2,), jnp.int32),
            pltpu.VMEM((rows_per_w // 2,), jnp.int32),
            pltpu.SemaphoreType.DMA,
        ],
    )
    def k5(y_hbm, posA_hbm, posB_hbm, y0_hbm, y1_hbm, r0, r1, i0v, i1v, sem):
        wid = lax.axis_index("s") * NC + lax.axis_index("c")
        half = rows_per_w // 2
        for hh in range(2):
            base = wid * rows_per_w + hh * half
            pltpu.sync_copy(posA_hbm.at[pl.ds(base, half)], i0v)
            pltpu.sync_copy(posB_hbm.at[pl.ds(base, half)], i1v)
            g0 = pltpu.async_copy(y_hbm.at[i0v], r0, sem)
            g1 = pltpu.async_copy(y_hbm.at[i1v], r1, sem)
            g0.wait()
            g1.wait()
            s0 = pltpu.async_copy(r0, y0_hbm.at[pl.ds(base, half)], sem)
            s1 = pltpu.async_copy(r1, y1_hbm.at[pl.ds(base, half)], sem)
            s0.wait()
            s1.wait()

    return k5


def kernel(x, gate_w, gate_bias, W1, b1, W2, b2, Ws1, bs1, Ws2, bs2):
    B, S, H = x.shape
    xf = x.reshape(-1, H)

    gw16 = jnp.pad(gate_w, ((0, 0), (0, 16 - NUM_EXPERTS)))
    gb16 = jnp.pad(gate_bias, (0, 16 - NUM_EXPERTS),
                   constant_values=-1e9).reshape(1, 16)

    nblk_r = T // RBT
    sA, sB, posA, posB, binfo, nact = pl.pallas_call(
        _route_body,
        grid=(nblk_r,),
        in_specs=[
            pl.BlockSpec((RBT, HIDDEN), lambda b: (b, 0)),
            pl.BlockSpec((HIDDEN, 16), lambda b: (0, 0)),
            pl.BlockSpec((1, 16), lambda b: (0, 0)),
        ],
        out_specs=[
            pl.BlockSpec((RBT, 1), lambda b: (b, 0)),
            pl.BlockSpec((RBT, 1), lambda b: (b, 0)),
            pl.BlockSpec((T, 1), lambda b: (0, 0)),
            pl.BlockSpec((T, 1), lambda b: (0, 0)),
            pl.BlockSpec((1, 128), lambda b: (0, 0)),
            pl.BlockSpec((1, 128), lambda b: (0, 0)),
        ],
        out_shape=[
            jax.ShapeDtypeStruct((T, 1), jnp.float32),
            jax.ShapeDtypeStruct((T, 1), jnp.float32),
            jax.ShapeDtypeStruct((T, 1), jnp.int32),
            jax.ShapeDtypeStruct((T, 1), jnp.int32),
            jax.ShapeDtypeStruct((1, 128), jnp.int32),
            jax.ShapeDtypeStruct((1, 128), jnp.int32),
        ],
        scratch_shapes=[pltpu.VMEM((1, 16), jnp.float32),
                        pltpu.VMEM((T, 1), jnp.int32),
                        pltpu.VMEM((T, 1), jnp.int32),
                        pltpu.VMEM((T, 1), jnp.float32),
                        pltpu.VMEM((T, 1), jnp.float32)],
    )(xf, gw16, gb16)

    pos0 = posA.reshape(T)
    pos1 = posB.reshape(T)

    xs = _make_sc_scatter()(xf, pos0, pos1)

    W1s = jnp.concatenate([W1, Ws1[None]], axis=0).astype(jnp.bfloat16)
    W1s = W1s.reshape((NUM_EXPERTS + 1) * HIDDEN, D_FF)
    W2s = jnp.concatenate([W2, Ws2[None]], axis=0).astype(jnp.bfloat16)
    W2s = W2s.reshape((NUM_EXPERTS + 1) * D_FF, HIDDEN)
    b1s = jnp.concatenate([b1, bs1[None]], axis=0).reshape(NUM_EXPERTS + 1, 1, D_FF)
    b2s = jnp.concatenate([b2, bs2[None]], axis=0).reshape(NUM_EXPERTS + 1, 1, HIDDEN)

    grid_spec = pltpu.PrefetchScalarGridSpec(
        num_scalar_prefetch=2,
        grid=(NBLK,),
        in_specs=[
            pl.BlockSpec((FBT, HIDDEN), lambda b, binfo, nact: (b, 0)),
            pl.BlockSpec((HIDDEN, D_FF), lambda b, binfo, nact: (binfo[b], 0)),
            pl.BlockSpec((1, 1, D_FF), lambda b, binfo, nact: (binfo[b], 0, 0)),
            pl.BlockSpec((D_FF, HIDDEN), lambda b, binfo, nact: (binfo[b], 0)),
            pl.BlockSpec((1, 1, HIDDEN), lambda b, binfo, nact: (binfo[b], 0, 0)),
        ],
        out_specs=pl.BlockSpec((FBT, HIDDEN), lambda b, binfo, nact: (b, 0)),
    )
    y = pl.pallas_call(
        _ffn_body,
        grid_spec=grid_spec,
        out_shape=jax.ShapeDtypeStruct((NSLOT, HIDDEN), jnp.float32),
    )(binfo.reshape(128), nact.reshape(128)[:1], xs, W1s, b1s, W2s, b2s)

    y0, y1 = _make_sc_gather()(y, pos0, pos1)

    out = pl.pallas_call(
        _combine_body,
        grid=(T // RBT,),
        in_specs=[
            pl.BlockSpec((RBT, HIDDEN), lambda t: (t, 0)),
            pl.BlockSpec((RBT, HIDDEN), lambda t: (t, 0)),
            pl.BlockSpec((RBT, HIDDEN), lambda t: (t, 0)),
            pl.BlockSpec((RBT, 1), lambda t: (t, 0)),
            pl.BlockSpec((RBT, 1), lambda t: (t, 0)),
        ],
        out_specs=pl.BlockSpec((RBT, HIDDEN), lambda t: (t, 0)),
        out_shape=jax.ShapeDtypeStruct((T, HIDDEN), jnp.float32),
    )(y0, y1, y, sA, sB)

    return out.reshape(B, S, H)
